# baseline (speedup 1.0000x reference)
import jax
import jax.numpy as jnp
from jax import lax
from jax.experimental import pallas as pl
from jax.experimental.pallas import tpu as pltpu
from jax.experimental.pallas import tpu_sc as plsc

DIM = 64
BATCH = 16384
N_FIELDS = 26
B_TOTAL = BATCH * N_FIELDS
N_EMB = 1_000_000

_info = plsc.get_sparse_core_info()
_NC, _NS = _info.num_cores, _info.num_subcores
NW = _NC * _NS
S_PER_W = BATCH // NW
SB = 128
N_SB = S_PER_W // SB
N_BLK = N_SB * N_FIELDS

TBLK = 512
T_GRID = (N_EMB + TBLK - 1) // TBLK


def _transpose_body(in_ref, out_ref):
    t = jnp.transpose(in_ref[...], (1, 0))
    out_ref[...] = jnp.concatenate([t, jnp.zeros_like(t)], axis=1)


def _emb_body(table_hbm, idx_hbm, out_hbm,
              idx_v, ch0, ch1, gb0, gb1, ob0, ob1, sg0, sg1, sw0, sw1):
    wid = lax.axis_index("s") * _NC + lax.axis_index("c")
    base = wid * (S_PER_W * N_FIELDS)
    pltpu.sync_copy(idx_hbm.at[pl.ds(base, S_PER_W * N_FIELDS)], idx_v)

    iota = lax.iota(jnp.int32, 16)
    gcols = [iota * N_FIELDS + (16 * N_FIELDS) * k for k in range(SB // 16)]
    trows = [iota + 16 * k for k in range(DIM // 16)]

    def blk_fs(b):
        f = lax.rem(b, N_FIELDS)
        s_tl = b // N_FIELDS
        return f, s_tl

    def build_chunk(b, ch):
        f, s_tl = blk_fs(b)
        p0 = s_tl * (SB * N_FIELDS) + f
        for k in range(SB // 16):
            v = plsc.load_gather(idx_v, [gcols[k] + p0])
            ch[pl.ds(16 * k, 16)] = v

    def fire(ch, gb, sem):
        pltpu.async_copy(table_hbm.at[ch], gb, sem)

    def drain_gather(gb, sem):
        pltpu.make_async_copy(table_hbm.at[pl.ds(0, SB)], gb, sem).wait()

    def transpose_blk(gb, ob):
        def body(t, carry):
            for u in range(4):
                sl = t * 4 + u
                colv = jnp.full((16,), sl, jnp.int32)
                for k in range(DIM // 16):
                    v = gb[sl, pl.ds(16 * k, 16)]
                    plsc.store_scatter(ob, [trows[k], colv], v)
            return carry
        lax.fori_loop(0, SB // 4, body, 0)

    def wb_start(b, ob, sem):
        f, s_tl = blk_fs(b)
        s0 = wid * S_PER_W + s_tl * SB
        pltpu.async_copy(ob, out_hbm.at[f, :, pl.ds(s0, SB)], sem)

    def wb_wait(b, ob, sem):
        f, s_tl = blk_fs(b)
        s0 = wid * S_PER_W + s_tl * SB
        pltpu.make_async_copy(ob, out_hbm.at[f, :, pl.ds(s0, SB)], sem).wait()

    build_chunk(0, ch0)
    fire(ch0, gb0, sg0)
    build_chunk(1, ch1)
    fire(ch1, gb1, sg1)

    drain_gather(gb0, sg0)
    transpose_blk(gb0, ob0)
    wb_start(0, ob0, sw0)
    build_chunk(2, ch0)
    fire(ch0, gb0, sg0)
    drain_gather(gb1, sg1)
    transpose_blk(gb1, ob1)
    wb_start(1, ob1, sw1)
    build_chunk(3, ch1)
    fire(ch1, gb1, sg1)

    def pair(i, carry):
        b = 2 * i
        drain_gather(gb0, sg0)
        wb_wait(b - 2, ob0, sw0)
        transpose_blk(gb0, ob0)
        wb_start(b, ob0, sw0)
        build_chunk(b + 2, ch0)
        fire(ch0, gb0, sg0)
        drain_gather(gb1, sg1)
        wb_wait(b - 1, ob1, sw1)
        transpose_blk(gb1, ob1)
        wb_start(b + 1, ob1, sw1)
        build_chunk(b + 3, ch1)
        fire(ch1, gb1, sg1)
        return carry

    lax.fori_loop(1, N_BLK // 2 - 1, pair, 0)

    b = N_BLK - 2
    drain_gather(gb0, sg0)
    wb_wait(b - 2, ob0, sw0)
    transpose_blk(gb0, ob0)
    wb_start(b, ob0, sw0)
    drain_gather(gb1, sg1)
    wb_wait(b - 1, ob1, sw1)
    transpose_blk(gb1, ob1)
    wb_start(b + 1, ob1, sw1)
    wb_wait(b, ob0, sw0)
    wb_wait(b + 1, ob1, sw1)


@jax.jit
def kernel(idx, weight):
    wT = weight.T
    table = pl.pallas_call(
        _transpose_body,
        grid=(T_GRID,),
        in_specs=[pl.BlockSpec((DIM, TBLK), lambda j: (0, j))],
        out_specs=pl.BlockSpec((TBLK, 128), lambda j: (j, 0)),
        out_shape=jax.ShapeDtypeStruct((N_EMB, 128), jnp.float32),
    )(wT)

    idx_flat = idx.reshape(-1).astype(jnp.int32)
    out = pl.kernel(
        _emb_body,
        out_type=jax.ShapeDtypeStruct((N_FIELDS, DIM, BATCH), jnp.float32),
        mesh=plsc.VectorSubcoreMesh(core_axis_name="c", subcore_axis_name="s"),
        scratch_types=[
            pltpu.VMEM((S_PER_W * N_FIELDS,), jnp.int32),
            pltpu.VMEM((SB,), jnp.int32),
            pltpu.VMEM((SB,), jnp.int32),
            pltpu.VMEM((SB, 128), jnp.float32),
            pltpu.VMEM((SB, 128), jnp.float32),
            pltpu.VMEM((DIM, SB), jnp.float32),
            pltpu.VMEM((DIM, SB), jnp.float32),
            pltpu.SemaphoreType.DMA,
            pltpu.SemaphoreType.DMA,
            pltpu.SemaphoreType.DMA,
            pltpu.SemaphoreType.DMA,
        ],
        compiler_params=pltpu.CompilerParams(
            use_tc_tiling_on_sc=True, needs_layout_passes=False),
    )(table, idx_flat)
    return out.transpose(2, 0, 1)
